# baseline (speedup 1.0000x reference)
import functools

import jax
import jax.numpy as jnp
from jax import lax
from jax.experimental import pallas as pl
from jax.experimental.pallas import tpu as pltpu
from jax.experimental.pallas import tpu_sc as plsc

B = 4
S = 2048
H = 768
N_TOK = B * S
NW = 32
SW = S // NW
KB = 32
NBLK = B * SW // KB
HV = H // 16
EPS = 1e-12
ROW_BYTES = KB * H * 4


def _row_base(wid, blk):
    h = blk // B
    b = blk - h * B
    return b * S + wid * SW + h * KB


def _sc_body(word_hbm, ent_hbm, pos_hbm, type_hbm, gamma_hbm, beta_hbm,
             idsw_hbm, idse_hbm, idst_hbm, out_hbm,
             idw, ide, ttv, posb, tyb, dtb, gb, bb, wbuf, ebuf,
             semw, seme, semo):
    wid = lax.axis_index("s") * 2 + lax.axis_index("c")

    pltpu.sync_copy(idsw_hbm.at[wid], idw)
    pltpu.sync_copy(idse_hbm.at[wid], ide)
    pltpu.sync_copy(idst_hbm.at[wid], ttv)
    pltpu.sync_copy(type_hbm, tyb)
    pltpu.sync_copy(gamma_hbm, gb)
    pltpu.sync_copy(beta_hbm, bb)

    for hh in range(HV):
        sl = pl.ds(hh * 16, 16)
        dtb[sl] = tyb[1, sl] - tyb[0, sl]

    def gather(blk, buf):
        cw = pltpu.async_copy(word_hbm.at[idw.at[blk]], wbuf.at[buf], semw)
        ce = pltpu.async_copy(ent_hbm.at[ide.at[blk]], ebuf.at[buf], seme)
        return cw, ce

    def wait_gather(blk, buf):
        pltpu.make_async_copy(word_hbm.at[idw.at[blk]], wbuf.at[buf], semw).wait()
        pltpu.make_async_copy(ent_hbm.at[ide.at[blk]], ebuf.at[buf], seme).wait()

    def out_slice(blk):
        return out_hbm.at[pl.ds(_row_base(wid, blk), KB)]

    def load_pos(blk):
        h = blk // B
        pltpu.sync_copy(pos_hbm.at[pl.ds(wid * SW + h * KB, KB)], posb)
        def addt0(t, c):
            for hh in range(HV):
                sl = pl.ds(hh * 16, 16)
                posb[t, sl] = posb[t, sl] + tyb[0, sl]
            return c
        lax.fori_loop(0, KB, addt0, 0)

    def compute(blk, buf):
        def row(t, c):
            g = t // 16
            lane = t - g * 16
            ttg = ttv[blk, pl.ds(g * 16, 16)].astype(jnp.float32)
            ttf = ttg.at[jnp.full((16,), lane, jnp.int32)].get(
                mode="promise_in_bounds")
            acc_s = jnp.zeros((16,), jnp.float32)
            acc_q = jnp.zeros((16,), jnp.float32)
            for hh in range(HV):
                sl = pl.ds(hh * 16, 16)
                x = wbuf[buf, t, sl] + ebuf[buf, t, sl]
                x = x + posb[t, sl] + ttf * dtb[sl]
                wbuf[buf, t, sl] = x
                acc_s = acc_s + x
                acc_q = acc_q + x * x
            for sh in (8, 4, 2, 1):
                idx = lax.iota(jnp.int32, 16) ^ sh
                acc_s = acc_s + acc_s.at[idx].get(mode="promise_in_bounds")
                acc_q = acc_q + acc_q.at[idx].get(mode="promise_in_bounds")
            muv = acc_s * (1.0 / H)
            vv = acc_q * (1.0 / H) - muv * muv + EPS
            iv = lax.bitcast_convert_type(vv, jnp.int32)
            iv = 0x5F3759DF - lax.shift_right_logical(iv, 1)
            y = lax.bitcast_convert_type(iv, jnp.float32)
            hv = 0.5 * vv
            for _ in range(4):
                y = y * (1.5 - hv * y * y)
            for hh in range(HV):
                sl = pl.ds(hh * 16, 16)
                x = wbuf[buf, t, sl]
                wbuf[buf, t, sl] = (x - muv) * y * gb[sl] + bb[sl]
            return c
        lax.fori_loop(0, KB, row, 0)

    load_pos(0)
    gather(0, 0)
    for blk in range(NBLK):
        buf = blk % 2
        if blk + 1 < NBLK:
            if blk >= 1:
                pltpu.make_async_copy(wbuf.at[1 - buf], out_slice(blk - 1), semo).wait()
            gather(blk + 1, 1 - buf)
        if blk > 0 and blk % B == 0:
            load_pos(blk)
        wait_gather(blk, buf)
        compute(blk, buf)
        pltpu.async_copy(wbuf.at[buf], out_slice(blk), semo)
    pltpu.make_async_copy(wbuf.at[0], out_slice(NBLK - 2), semo).wait()
    pltpu.make_async_copy(wbuf.at[1], out_slice(NBLK - 1), semo).wait()


_sc_full = functools.partial(
    pl.kernel,
    out_type=jax.ShapeDtypeStruct((N_TOK, H), jnp.float32),
    mesh=plsc.VectorSubcoreMesh(core_axis_name="c", subcore_axis_name="s"),
    scratch_types=[
        pltpu.VMEM((NBLK, KB), jnp.int32),
        pltpu.VMEM((NBLK, KB), jnp.int32),
        pltpu.VMEM((NBLK, KB), jnp.int32),
        pltpu.VMEM((KB, H), jnp.float32),
        pltpu.VMEM((2, H), jnp.float32),
        pltpu.VMEM((H,), jnp.float32),
        pltpu.VMEM((H,), jnp.float32),
        pltpu.VMEM((H,), jnp.float32),
        pltpu.VMEM((2, KB, H), jnp.float32),
        pltpu.VMEM((2, KB, H), jnp.float32),
        pltpu.SemaphoreType.DMA,
        pltpu.SemaphoreType.DMA,
        pltpu.SemaphoreType.DMA,
    ],
)(_sc_body)


def _permute_ids(x):
    return (x.astype(jnp.int32)
            .reshape(B, NW, SW // KB, KB)
            .transpose(1, 2, 0, 3)
            .reshape(NW, NBLK, KB))


def kernel(input_ids, token_type_ids, entity_ids, word_table, pos_table,
           type_table, entity_table, gamma, beta):
    idsw = _permute_ids(input_ids)
    idse = _permute_ids(entity_ids)
    idst = _permute_ids(token_type_ids)
    out = _sc_full(word_table, entity_table, pos_table, type_table,
                   gamma, beta, idsw, idse, idst)
    return out.reshape(B, S, H)


# speedup vs baseline: 2.4209x; 2.4209x over previous
import functools

import jax
import jax.numpy as jnp
from jax import lax
from jax.experimental import pallas as pl
from jax.experimental.pallas import tpu as pltpu
from jax.experimental.pallas import tpu_sc as plsc

B = 4
S = 2048
H = 768
N_TOK = B * S
NW = 32
TOK_PER_W = N_TOK // NW
KB = 64
NB = TOK_PER_W // KB
HV = H // 16
EPS = 1e-12

BS_TC = 1024
S_BLKS = S // BS_TC


def _sc_gather_sum_body(word_hbm, ent_hbm, ids_hbm, eids_hbm, out_hbm,
                        idw, ide, wbuf, ebuf, semw, seme):
    wid = lax.axis_index("s") * 2 + lax.axis_index("c")
    base = wid * TOK_PER_W
    pltpu.sync_copy(ids_hbm.at[pl.ds(base, TOK_PER_W)], idw)
    pltpu.sync_copy(eids_hbm.at[pl.ds(base, TOK_PER_W)], ide)

    def do_block(b, carry):
        cw = pltpu.async_copy(word_hbm.at[idw.at[pl.ds(b * KB, KB)]], wbuf, semw)
        ce = pltpu.async_copy(ent_hbm.at[ide.at[pl.ds(b * KB, KB)]], ebuf, seme)
        cw.wait()
        ce.wait()

        def addrow(t, c2):
            for h in range(HV):
                sl = pl.ds(h * 16, 16)
                wbuf[t, sl] = wbuf[t, sl] + ebuf[t, sl]
            return c2

        lax.fori_loop(0, KB, addrow, 0)
        pltpu.sync_copy(wbuf, out_hbm.at[pl.ds(base + b * KB, KB)])
        return carry

    lax.fori_loop(0, NB, do_block, 0)


_sc_gather_sum = functools.partial(
    pl.kernel,
    out_type=jax.ShapeDtypeStruct((N_TOK, H), jnp.float32),
    mesh=plsc.VectorSubcoreMesh(core_axis_name="c", subcore_axis_name="s"),
    scratch_types=[
        pltpu.VMEM((TOK_PER_W,), jnp.int32),
        pltpu.VMEM((TOK_PER_W,), jnp.int32),
        pltpu.VMEM((KB, H), jnp.float32),
        pltpu.VMEM((KB, H), jnp.float32),
        pltpu.SemaphoreType.DMA,
        pltpu.SemaphoreType.DMA,
    ],
)(_sc_gather_sum_body)


def _ln_body(sum_ref, pos_ref, ttf_ref, type_ref, gamma_ref, beta_ref, out_ref):
    t0 = type_ref[0:1, :]
    t1 = type_ref[1:2, :]
    x = sum_ref[...] + pos_ref[...] + t0 + ttf_ref[...] * (t1 - t0)
    mu = jnp.mean(x, axis=-1, keepdims=True)
    xc = x - mu
    var = jnp.mean(xc * xc, axis=-1, keepdims=True)
    r = lax.rsqrt(var + EPS)
    out_ref[...] = xc * r * gamma_ref[...] + beta_ref[...]


def _tc_layernorm(ssum, pos_table, ttf, type_table, gamma, beta):
    nb = S // BS_TC
    return pl.pallas_call(
        _ln_body,
        grid=(S_BLKS, B),
        in_specs=[
            pl.BlockSpec((BS_TC, H), lambda s, b: (b * nb + s, 0)),
            pl.BlockSpec((BS_TC, H), lambda s, b: (s, 0)),
            pl.BlockSpec((BS_TC, 1), lambda s, b: (b * nb + s, 0)),
            pl.BlockSpec((2, H), lambda s, b: (0, 0)),
            pl.BlockSpec((1, H), lambda s, b: (0, 0)),
            pl.BlockSpec((1, H), lambda s, b: (0, 0)),
        ],
        out_specs=pl.BlockSpec((BS_TC, H), lambda s, b: (b * nb + s, 0)),
        out_shape=jax.ShapeDtypeStruct((N_TOK, H), jnp.float32),
    )(ssum, pos_table, ttf, type_table, gamma, beta)


def kernel(input_ids, token_type_ids, entity_ids, word_table, pos_table,
           type_table, entity_table, gamma, beta):
    ids = input_ids.reshape(-1).astype(jnp.int32)
    eids = entity_ids.reshape(-1).astype(jnp.int32)
    ttf = token_type_ids.reshape(-1, 1).astype(jnp.float32)
    ssum = _sc_gather_sum(word_table, entity_table, ids, eids)
    out = _tc_layernorm(ssum, pos_table, ttf, type_table,
                        gamma.reshape(1, H), beta.reshape(1, H))
    return out.reshape(B, S, H)
